# Initial kernel scaffold; baseline (speedup 1.0000x reference)
import jax
import jax.numpy as jnp
from jax.experimental import pallas as pl


def kernel(x, weight):
    raise NotImplementedError("write your pallas kernel here")


# speedup vs baseline: 1.8430x; 1.8430x over previous
import functools

import jax
import jax.numpy as jnp
from jax import lax
from jax.experimental import pallas as pl
from jax.experimental.pallas import tpu as pltpu
from jax.experimental.pallas import tpu_sc as plsc

NUM_EMB = 1_000_000
DIM = 64
IDX_W = 128
ROWS_PER_STEP = 8


@functools.cache
def _build(n_idx_rows: int):
    info = plsc.get_sparse_core_info()
    nc, ns = info.num_cores, info.num_subcores
    nw = nc * ns
    rows_per_w = n_idx_rows // nw
    n_steps = rows_per_w // ROWS_PER_STEP
    chunk = ROWS_PER_STEP * IDX_W

    mesh = plsc.VectorSubcoreMesh(core_axis_name="c", subcore_axis_name="s")

    @functools.partial(
        pl.kernel,
        mesh=mesh,
        out_type=jax.ShapeDtypeStruct((n_idx_rows * IDX_W, DIM), jnp.float32),
        scratch_types=[
            pltpu.VMEM((ROWS_PER_STEP, IDX_W), jnp.int32),
            pltpu.VMEM((chunk, DIM), jnp.float32),
            pltpu.SemaphoreType.DMA,
        ],
        compiler_params=pltpu.CompilerParams(use_tc_tiling_on_sc=False),
    )
    def k(idx_hbm, table_hbm, out_hbm, idx_v, rows_v, sem):
        wid = lax.axis_index("s") * nc + lax.axis_index("c")
        row0 = wid * rows_per_w

        def step(i, carry):
            roff = row0 + i * ROWS_PER_STEP
            pltpu.sync_copy(idx_hbm.at[pl.ds(roff, ROWS_PER_STEP)], idx_v)
            copies = [
                pltpu.async_copy(
                    table_hbm.at[idx_v.at[j]],
                    rows_v.at[pl.ds(j * IDX_W, IDX_W)],
                    sem,
                )
                for j in range(ROWS_PER_STEP)
            ]
            for c in copies:
                c.wait()
            pltpu.sync_copy(rows_v, out_hbm.at[pl.ds(roff * IDX_W, chunk)])
            return carry

        lax.fori_loop(0, n_steps, step, 0)

    return k


def kernel(x, weight):
    b, h = x.shape
    idx = x.astype(jnp.int32).reshape(-1, IDX_W)
    out = _build(idx.shape[0])(idx, weight)
    return out.reshape(b, h, DIM)


# speedup vs baseline: 1.8580x; 1.0081x over previous
import functools

import jax
import jax.numpy as jnp
from jax import lax
from jax.experimental import pallas as pl
from jax.experimental.pallas import tpu as pltpu
from jax.experimental.pallas import tpu_sc as plsc

DIM = 64
IDX_W = 128
R = 4
CHUNK = R * IDX_W


@functools.cache
def _build(n_idx_rows: int):
    info = plsc.get_sparse_core_info()
    nc, ns = info.num_cores, info.num_subcores
    nw = nc * ns
    rows_per_w = n_idx_rows // nw
    n_chunks = rows_per_w // R
    assert n_chunks * R == rows_per_w and n_chunks % 2 == 0 and n_chunks >= 4

    mesh = plsc.VectorSubcoreMesh(core_axis_name="c", subcore_axis_name="s")

    @functools.partial(
        pl.kernel,
        mesh=mesh,
        out_type=jax.ShapeDtypeStruct((n_idx_rows * IDX_W, DIM), jnp.float32),
        scratch_types=[
            pltpu.VMEM((2, R, IDX_W), jnp.int32),
            pltpu.VMEM((2, CHUNK, DIM), jnp.float32),
            pltpu.SemaphoreType.DMA,
            pltpu.SemaphoreType.DMA,
        ],
        compiler_params=pltpu.CompilerParams(use_tc_tiling_on_sc=False),
    )
    def k(idx_hbm, table_hbm, out_hbm, idx_v, rows_v, gsem, osem):
        wid = lax.axis_index("s") * nc + lax.axis_index("c")
        row0 = wid * rows_per_w

        def fire(g, b):
            pltpu.sync_copy(idx_hbm.at[pl.ds(row0 + g * R, R)], idx_v.at[b])
            for j in range(R):
                pltpu.async_copy(
                    table_hbm.at[idx_v.at[b, j]],
                    rows_v.at[b, pl.ds(j * IDX_W, IDX_W)],
                    gsem,
                )

        def drain_gather(b):
            for j in range(R):
                pltpu.make_async_copy(
                    table_hbm.at[idx_v.at[b, j]],
                    rows_v.at[b, pl.ds(j * IDX_W, IDX_W)],
                    gsem,
                ).wait()

        def out_slice(g):
            return out_hbm.at[pl.ds((row0 + g * R) * IDX_W, CHUNK)]

        def fire_out(g, b):
            pltpu.async_copy(rows_v.at[b], out_slice(g), osem)

        def drain_out(g, b):
            pltpu.make_async_copy(rows_v.at[b], out_slice(g), osem).wait()

        fire(0, 0)
        fire(1, 1)
        drain_gather(0)
        fire_out(0, 0)

        @pl.loop(0, (n_chunks - 2) // 2)
        def _pair(p):
            g = 1 + 2 * p
            for db in range(2):
                b = (1 + db) % 2
                drain_out(g + db - 1, 1 - b)
                fire(g + db + 1, 1 - b)
                drain_gather(b)
                fire_out(g + db, b)

        drain_out(n_chunks - 2, 0)
        drain_gather(1)
        fire_out(n_chunks - 1, 1)
        drain_out(n_chunks - 1, 1)

    return k


def kernel(x, weight):
    b, h = x.shape
    idx = x.astype(jnp.int32).reshape(-1, IDX_W)
    out = _build(idx.shape[0])(idx, weight)
    return out.reshape(b, h, DIM)
